# baseline (speedup 1.0000x reference)
import functools

import jax
import jax.numpy as jnp
from jax import lax
from jax.experimental import pallas as pl
from jax.experimental.pallas import tpu as pltpu
from jax.experimental.pallas import tpu_sc as plsc

D_MODEL = 2048
NC = 2
NS = 16
NW = NC * NS
CH = 4
NBUF = 8


@functools.lru_cache(maxsize=None)
def _make_gather(b_total):
    b_per_w = b_total // NW
    nchunk = b_per_w // CH
    assert nchunk % NBUF == 0
    H = nchunk // NBUF

    mesh = plsc.VectorSubcoreMesh(core_axis_name="c", subcore_axis_name="s")

    scratch = (
        [pltpu.VMEM((nchunk, CH), jnp.int32)]
        + [pltpu.VMEM((CH, D_MODEL), jnp.float32)] * NBUF
        + [pltpu.SemaphoreType.DMA] * (2 * NBUF)
    )

    @functools.partial(
        pl.kernel,
        mesh=mesh,
        out_type=jax.ShapeDtypeStruct((b_total, D_MODEL), jnp.float32),
        scratch_types=scratch,
    )
    def gather_kernel(table, idx, out, idx_v, *bufs_and_sems):
        rows = bufs_and_sems[:NBUF]
        gsem = bufs_and_sems[NBUF:2 * NBUF]
        osem = bufs_and_sems[2 * NBUF:]

        wid = lax.axis_index("s") * NC + lax.axis_index("c")
        base = wid * b_per_w
        pltpu.sync_copy(idx.at[wid], idx_v)

        def start_g(b, c):
            pltpu.async_copy(table.at[idx_v.at[c]], rows[b], gsem[b])

        def wait_g(b):
            pltpu.make_async_copy(table.at[pl.ds(0, CH)], rows[b], gsem[b]).wait()

        def start_o(b, c):
            pltpu.async_copy(rows[b], out.at[pl.ds(base + c * CH, CH)], osem[b])

        def wait_o(b):
            pltpu.make_async_copy(rows[b], out.at[pl.ds(base, CH)], osem[b]).wait()

        for j in range(NBUF - 1):
            start_g(j, j)

        def step(h, carry):
            for b in range(NBUF):
                c = h * NBUF + b
                wait_g(b)
                start_o(b, c)
                nb = (b - 1) % NBUF
                ng = c + NBUF - 1
                if b == 0:
                    @pl.when(h > 0)
                    def _():
                        wait_o(nb)
                else:
                    wait_o(nb)

                @pl.when(ng < nchunk)
                def _():
                    start_g(nb, ng)
            return carry

        lax.fori_loop(0, H, step, 0)
        wait_o((nchunk - 1) % NBUF)

    return gather_kernel


@jax.jit
def _impl(position_ids, wpe):
    b, s = position_ids.shape
    b_total = b * s
    idx = position_ids.astype(jnp.int32).reshape(NW, b_total // NW // CH, CH)
    out = _make_gather(b_total)(wpe, idx)
    return out.reshape(b, s, D_MODEL)


def kernel(position_ids, wpe):
    return _impl(position_ids, wpe)


# speedup vs baseline: 1.0057x; 1.0057x over previous
import functools

import jax
import jax.numpy as jnp
from jax import lax
from jax.experimental import pallas as pl
from jax.experimental.pallas import tpu as pltpu
from jax.experimental.pallas import tpu_sc as plsc

D_MODEL = 2048
NC = 2
NS = 16
NW = NC * NS
CH = 8
NBUF = 4


@functools.lru_cache(maxsize=None)
def _make_gather(b_total):
    b_per_w = b_total // NW
    nchunk = b_per_w // CH
    assert nchunk % NBUF == 0
    H = nchunk // NBUF

    mesh = plsc.VectorSubcoreMesh(core_axis_name="c", subcore_axis_name="s")

    scratch = (
        [pltpu.VMEM((nchunk, CH), jnp.int32)]
        + [pltpu.VMEM((CH, D_MODEL), jnp.float32)] * NBUF
        + [pltpu.SemaphoreType.DMA] * (2 * NBUF)
    )

    @functools.partial(
        pl.kernel,
        mesh=mesh,
        out_type=jax.ShapeDtypeStruct((b_total, D_MODEL), jnp.float32),
        scratch_types=scratch,
    )
    def gather_kernel(table, idx, out, idx_v, *bufs_and_sems):
        rows = bufs_and_sems[:NBUF]
        gsem = bufs_and_sems[NBUF:2 * NBUF]
        osem = bufs_and_sems[2 * NBUF:]

        wid = lax.axis_index("s") * NC + lax.axis_index("c")
        pltpu.sync_copy(idx.at[wid], idx_v)

        def start_g(b, c):
            pltpu.async_copy(table.at[idx_v.at[c]], rows[b], gsem[b])

        def wait_g(b):
            pltpu.make_async_copy(table.at[pl.ds(0, CH)], rows[b], gsem[b]).wait()

        def start_o(b, c):
            pltpu.async_copy(rows[b], out.at[pl.ds((c * NW + wid) * CH, CH)], osem[b])

        def wait_o(b):
            pltpu.make_async_copy(rows[b], out.at[pl.ds(0, CH)], osem[b]).wait()

        for j in range(NBUF - 1):
            start_g(j, j)

        def step(h, carry):
            for b in range(NBUF):
                c = h * NBUF + b
                wait_g(b)
                start_o(b, c)
                nb = (b - 1) % NBUF
                ng = c + NBUF - 1
                if b == 0:
                    @pl.when(h > 0)
                    def _():
                        wait_o(nb)
                else:
                    wait_o(nb)

                @pl.when(ng < nchunk)
                def _():
                    start_g(nb, ng)
            return carry

        lax.fori_loop(0, H, step, 0)
        wait_o((nchunk - 1) % NBUF)

    return gather_kernel


@jax.jit
def _impl(position_ids, wpe):
    b, s = position_ids.shape
    b_total = b * s
    nchunk = b_total // NW // CH
    idx = (
        position_ids.astype(jnp.int32)
        .reshape(nchunk, NW, CH)
        .transpose(1, 0, 2)
    )
    out = _make_gather(b_total)(wpe, idx)
    return out.reshape(b, s, D_MODEL)


def kernel(position_ids, wpe):
    return _impl(position_ids, wpe)
